# baseline (speedup 1.0000x reference)
import functools

import jax
import jax.numpy as jnp
from jax import lax
from jax.experimental import pallas as pl
from jax.experimental.pallas import tpu as pltpu
from jax.experimental.pallas import tpu_sc as plsc

_VOCAB = 1000
_B, _L = 1024, 50
_K = 4
_BC = _B // _K
_NC, _NS = 2, 16
_NW = _NC * _NS
_BPW = _BC // _NW
_NBLK = _VOCAB // 128
_TAIL = _VOCAB - 128 * _NBLK
_LP = 56
_SLAB = _BPW * 8 * _LP


def _make_gather():
    mesh = plsc.VectorSubcoreMesh(core_axis_name="c", subcore_axis_name="s")

    @functools.partial(
        pl.kernel,
        mesh=mesh,
        out_type=jax.ShapeDtypeStruct((_BC, _L, _VOCAB), jnp.float32),
        scratch_types=[
            pltpu.VMEM((_SLAB,), jnp.int32),
            pltpu.VMEM((_L, 128 * _NBLK), jnp.float32),
            pltpu.VMEM((_L, 128 * _NBLK), jnp.float32),
            pltpu.VMEM((_L, 128), jnp.float32),
            pltpu.VMEM((_L, 128), jnp.float32),
        ] + [pltpu.SemaphoreType.DMA] * 8,
    )
    def gather_kernel(idxm_hbm, table_hbm, out_hbm, slab, bufa, bufb,
                      taila, tailb, ga, gb, gta, gtb, wa, wb, twa, twb):
        wid = lax.axis_index("s") * _NC + lax.axis_index("c")
        b0 = wid * _BPW
        pltpu.sync_copy(idxm_hbm.at[pl.ds(b0 * 8 * _LP, _SLAB)], slab)

        sets = ((bufa, taila, ga, gta, wa, twa),
                (bufb, tailb, gb, gtb, wb, twb))

        def start_gathers(k, buf, tail, g, gt):
            base = k * 8 * _LP
            for c in range(_NBLK):
                pltpu.async_copy(
                    table_hbm.at[slab.at[pl.ds(base + c * _LP, _L)]],
                    buf.at[:, pl.ds(c * 128, 128)], g)
            pltpu.async_copy(
                table_hbm.at[slab.at[pl.ds(base + _NBLK * _LP, _L)]],
                tail, gt)

        def complete(k, buf, tail, g, gt, w, tw):
            b = b0 + k
            pltpu.make_async_copy(
                table_hbm.at[slab.at[pl.ds(0, _L)]], tail, gt).wait()
            for r in range(_L):
                pltpu.async_copy(
                    tail.at[r, pl.ds(0, _TAIL)],
                    out_hbm.at[b, r, pl.ds(128 * _NBLK, _TAIL)], tw)
            for c in range(_NBLK):
                pltpu.make_async_copy(
                    table_hbm.at[slab.at[pl.ds(0, _L)]],
                    buf.at[:, pl.ds(c * 128, 128)], g).wait()
            pltpu.async_copy(
                buf, out_hbm.at[b, :, pl.ds(0, 128 * _NBLK)], w)

        def wait_outputs(buf, tail, w, tw):
            pltpu.make_async_copy(
                buf, out_hbm.at[b0, :, pl.ds(0, 128 * _NBLK)], w).wait()
            for r in range(_L):
                pltpu.make_async_copy(
                    tail.at[r, pl.ds(0, _TAIL)],
                    out_hbm.at[b0, r, pl.ds(128 * _NBLK, _TAIL)], tw).wait()

        start_gathers(0, bufa, taila, ga, gta)
        start_gathers(1, bufb, tailb, gb, gtb)

        def body(t, carry):
            k = 2 * t
            for j, (buf, tail, g, gt, w, tw) in enumerate(sets):
                complete(k + j, buf, tail, g, gt, w, tw)
            for j, (buf, tail, g, gt, w, tw) in enumerate(sets):
                wait_outputs(buf, tail, w, tw)
                start_gathers(k + 2 + j, buf, tail, g, gt)
            return carry

        lax.fori_loop(0, _BPW // 2 - 1, body, 0)

        for j, (buf, tail, g, gt, w, tw) in enumerate(sets):
            complete(_BPW - 2 + j, buf, tail, g, gt, w, tw)
        for buf, tail, g, gt, w, tw in sets:
            wait_outputs(buf, tail, w, tw)

    return gather_kernel


_gather = _make_gather()


@jax.jit
def kernel(idx, logits):
    table_p = jnp.pad(logits, ((0, 0), (0, 24))).reshape(8 * _VOCAB, 128)
    idxm = (idx * 8)[:, None, :] + jnp.arange(8, dtype=idx.dtype)[None, :, None]
    idxm = jnp.pad(idxm, ((0, 0), (0, 0), (0, _LP - _L))).reshape(_B, -1)
    out = jnp.zeros((_B, _L, _VOCAB), jnp.float32)
    for i in range(_K):
        chunk = _gather(idxm[i * _BC:(i + 1) * _BC].reshape(-1), table_p)
        out = lax.dynamic_update_slice(out, chunk, (i * _BC, 0, 0))
    return out


# speedup vs baseline: 1.3485x; 1.3485x over previous
import functools

import jax
import jax.numpy as jnp
from jax import lax
from jax.experimental import pallas as pl
from jax.experimental.pallas import tpu as pltpu
from jax.experimental.pallas import tpu_sc as plsc

_VOCAB = 1000
_B, _L = 1024, 50
_NC, _NS = 2, 16
_NW = _NC * _NS
_BPW = _B // _NW
_NBLK = _VOCAB // 128
_TAIL = _VOCAB - 128 * _NBLK
_LP = 56
_SLAB = _BPW * _LP


def _make_gather():
    mesh = plsc.VectorSubcoreMesh(core_axis_name="c", subcore_axis_name="s")

    @functools.partial(
        pl.kernel,
        mesh=mesh,
        out_type=jax.ShapeDtypeStruct((_B, _L, _VOCAB), jnp.float32),
        scratch_types=[
            pltpu.VMEM((_SLAB,), jnp.int32),
            pltpu.VMEM((_L, 8, 128), jnp.float32),
            pltpu.VMEM((_L, 8, 128), jnp.float32),
        ] + [pltpu.SemaphoreType.DMA] * 6,
    )
    def gather_kernel(idxp_hbm, table_hbm, out_hbm, slab, bufa, bufb,
                      ga, gb, wa, wb, twa, twb):
        wid = lax.axis_index("s") * _NC + lax.axis_index("c")
        b0 = wid * _BPW
        pltpu.sync_copy(idxp_hbm.at[pl.ds(b0 * _LP, _SLAB)], slab)

        sets = ((bufa, ga, wa, twa), (bufb, gb, wb, twb))

        def start_gather(k, buf, g):
            pltpu.async_copy(
                table_hbm.at[slab.at[pl.ds(k * _LP, _L)]], buf, g)

        def complete(k, buf, g, w, tw):
            b = b0 + k
            pltpu.make_async_copy(
                table_hbm.at[slab.at[pl.ds(0, _L)]], buf, g).wait()
            for c in range(_NBLK):
                pltpu.async_copy(
                    buf.at[:, c, :],
                    out_hbm.at[b, :, pl.ds(c * 128, 128)], w)
            for r in range(_L):
                pltpu.async_copy(
                    buf.at[r, _NBLK, pl.ds(0, _TAIL)],
                    out_hbm.at[b, r, pl.ds(128 * _NBLK, _TAIL)], tw)

        def wait_outputs(buf, w, tw):
            for c in range(_NBLK):
                pltpu.make_async_copy(
                    buf.at[:, c, :],
                    out_hbm.at[b0, :, pl.ds(c * 128, 128)], w).wait()
            for r in range(_L):
                pltpu.make_async_copy(
                    buf.at[r, _NBLK, pl.ds(0, _TAIL)],
                    out_hbm.at[b0, r, pl.ds(128 * _NBLK, _TAIL)], tw).wait()

        start_gather(0, bufa, ga)
        start_gather(1, bufb, gb)

        def body(t, carry):
            k = 2 * t
            for j, (buf, g, w, tw) in enumerate(sets):
                complete(k + j, buf, g, w, tw)
            for j, (buf, g, w, tw) in enumerate(sets):
                wait_outputs(buf, w, tw)
                start_gather(k + 2 + j, buf, g)
            return carry

        lax.fori_loop(0, _BPW // 2 - 1, body, 0)

        for j, (buf, g, w, tw) in enumerate(sets):
            complete(_BPW - 2 + j, buf, g, w, tw)
        for buf, g, w, tw in sets:
            wait_outputs(buf, w, tw)

    return gather_kernel


_gather = _make_gather()


@jax.jit
def kernel(idx, logits):
    table_p = jnp.pad(logits, ((0, 0), (0, 24))).reshape(_VOCAB, 8, 128)
    idxp = jnp.pad(idx, ((0, 0), (0, _LP - _L))).reshape(-1)
    return _gather(idxp, table_p)
